# Initial kernel scaffold; baseline (speedup 1.0000x reference)
import jax
import jax.numpy as jnp
from jax.experimental import pallas as pl


def kernel(x, adj, W_node_mu1, W_node_mu2, W_node_var1, W_node_var2, W_attr_mu1, b_attr_mu1, W_attr_mu_fc, b_attr_mu_fc, W_attr_var1, b_attr_var1, W_attr_var_fc, b_attr_var_fc):
    raise NotImplementedError("write your pallas kernel here")


# speedup vs baseline: 1.6749x; 1.6749x over previous
import jax
import jax.numpy as jnp
from jax.experimental import pallas as pl
from jax.experimental.pallas import tpu as pltpu

_NOISE = 5
_S = 2


def _prologue_body(x_ref, wa_ref, nn_ref, wnn_ref, wb_ref, an_ref, wna_ref,
                   bmu1_ref, wmufc_ref, bmufc_ref, bvar1_ref, wvarfc_ref,
                   bvarfc_ref, eps_attr_ref, pcat_ref, attrz_ref):
    out = pcat_ref.shape[1] // 2
    x = x_ref[...]
    pcat = jnp.dot(x, wa_ref[...], preferred_element_type=jnp.float32)
    pcat += jnp.dot(nn_ref[...], wnn_ref[...], preferred_element_type=jnp.float32)
    pcat_ref[...] = pcat
    acc = jax.lax.dot_general(x, wb_ref[...], (((0,), (0,)), ((), ())),
                              preferred_element_type=jnp.float32)
    pre_mu = (acc[:, :out] + bmu1_ref[...]
              + jnp.dot(an_ref[...], wna_ref[...],
                        preferred_element_type=jnp.float32))
    pre_var = acc[:, out:] + bvar1_ref[...]
    attr_mu = jnp.dot(jnp.tanh(pre_mu), wmufc_ref[...],
                      preferred_element_type=jnp.float32) + bmufc_ref[...]
    attr_logv = jnp.dot(jnp.tanh(pre_var), wvarfc_ref[...],
                        preferred_element_type=jnp.float32) + bvarfc_ref[...]
    attrz_ref[...] = attr_mu + eps_attr_ref[...] * jnp.exp(0.5 * attr_logv)


def _layer1_body(adj_ref, p_ref, w2_ref, q_ref):
    h = jnp.maximum(
        jnp.dot(adj_ref[...], p_ref[...], preferred_element_type=jnp.float32),
        0.0)
    q_ref[...] = jnp.dot(h, w2_ref[...], preferred_element_type=jnp.float32)


def _layer2_body(adj_ref, q_ref, eps_ref, attrz_ref, out_ref):
    out = q_ref.shape[1] // 2
    o = jnp.dot(adj_ref[...], q_ref[...], preferred_element_type=jnp.float32)
    z = o[:, :out] + eps_ref[...] * jnp.exp(0.5 * o[:, out:])
    out_ref[...] = jax.lax.dot_general(z, attrz_ref[...],
                                       (((1,), (1,)), ((), ())),
                                       preferred_element_type=jnp.float32)


def kernel(x, adj, W_node_mu1, W_node_mu2, W_node_var1, W_node_var2,
           W_attr_mu1, b_attr_mu1, W_attr_mu_fc, b_attr_mu_fc,
           W_attr_var1, b_attr_var1, W_attr_var_fc, b_attr_var_fc):
    n = adj.shape[0]
    f = x.shape[1]
    out = W_node_mu2.shape[0]
    f32 = jnp.float32

    nk = jax.random.key(123)
    nks = jax.random.split(nk, 4)
    node_noise = jax.random.bernoulli(
        nks[0], 0.5, (n, _S, _NOISE)).astype(f32)[:, 0, :]
    attr_noise = jax.random.bernoulli(
        nks[1], 0.5, (f, _S, _NOISE)).astype(f32)[:, 0, :]
    eps_node = jax.random.normal(nks[2], (n, 1, out), f32)[:, 0, :]
    eps_attr = jax.random.normal(nks[3], (f, 1, out), f32)[:, 0, :]

    wa = jnp.concatenate([W_node_mu1[_NOISE:], W_node_var1], axis=1)
    wnn = jnp.zeros((8, 2 * out), f32).at[:_NOISE, :out].set(W_node_mu1[:_NOISE])
    nn_pad = jnp.zeros((n, 8), f32).at[:, :_NOISE].set(node_noise)
    wb = jnp.concatenate([W_attr_mu1[_NOISE:], W_attr_var1], axis=1)
    wna = jnp.zeros((8, out), f32).at[:_NOISE].set(W_attr_mu1[:_NOISE])
    an_pad = jnp.zeros((f, 8), f32).at[:, :_NOISE].set(attr_noise)
    w2 = (jnp.zeros((2 * out, 2 * out), f32)
          .at[:out, :out].set(W_node_mu2)
          .at[out:, out:].set(W_node_var2))

    pcat, attr_z = pl.pallas_call(
        _prologue_body,
        out_shape=[jax.ShapeDtypeStruct((n, 2 * out), f32),
                   jax.ShapeDtypeStruct((f, out), f32)],
    )(x, wa, nn_pad, wnn, wb, an_pad, wna,
      b_attr_mu1.reshape(1, -1), W_attr_mu_fc, b_attr_mu_fc.reshape(1, -1),
      b_attr_var1.reshape(1, -1), W_attr_var_fc, b_attr_var_fc.reshape(1, -1),
      eps_attr)

    bm = 400
    qcat = pl.pallas_call(
        _layer1_body,
        grid=(n // bm,),
        in_specs=[pl.BlockSpec((bm, n), lambda i: (i, 0)),
                  pl.BlockSpec((n, 2 * out), lambda i: (0, 0)),
                  pl.BlockSpec((2 * out, 2 * out), lambda i: (0, 0))],
        out_specs=pl.BlockSpec((bm, 2 * out), lambda i: (i, 0)),
        out_shape=jax.ShapeDtypeStruct((n, 2 * out), f32),
        compiler_params=pltpu.CompilerParams(
            dimension_semantics=("parallel",)),
    )(adj, pcat, w2)

    recon = pl.pallas_call(
        _layer2_body,
        grid=(n // bm,),
        in_specs=[pl.BlockSpec((bm, n), lambda i: (i, 0)),
                  pl.BlockSpec((n, 2 * out), lambda i: (0, 0)),
                  pl.BlockSpec((bm, out), lambda i: (i, 0)),
                  pl.BlockSpec((f, out), lambda i: (0, 0))],
        out_specs=pl.BlockSpec((bm, f), lambda i: (i, 0)),
        out_shape=jax.ShapeDtypeStruct((n, f), f32),
        compiler_params=pltpu.CompilerParams(
            dimension_semantics=("parallel",)),
    )(adj, qcat, eps_node, attr_z)

    return recon


# speedup vs baseline: 1.7354x; 1.0362x over previous
import jax
import jax.numpy as jnp
from jax.experimental import pallas as pl
from jax.experimental.pallas import tpu as pltpu

_NOISE = 5
_S = 2


def _prologue_body(x_ref, wa_ref, nn_ref, wnn_ref, wb_ref, an_ref, wna_ref,
                   bmu1_ref, wmufc_ref, bmufc_ref, bvar1_ref, wvarfc_ref,
                   bvarfc_ref, eps_attr_ref, pcat_ref, attrz_ref):
    out = pcat_ref.shape[1] // 2
    x = x_ref[...]
    pcat = jnp.dot(x, wa_ref[...], preferred_element_type=jnp.float32)
    pcat += jnp.dot(nn_ref[...], wnn_ref[...], preferred_element_type=jnp.float32)
    pcat_ref[...] = pcat.astype(pcat_ref.dtype)
    acc = jax.lax.dot_general(x, wb_ref[...], (((0,), (0,)), ((), ())),
                              preferred_element_type=jnp.float32)
    pre_mu = (acc[:, :out] + bmu1_ref[...]
              + jnp.dot(an_ref[...], wna_ref[...],
                        preferred_element_type=jnp.float32))
    pre_var = acc[:, out:] + bvar1_ref[...]
    attr_mu = jnp.dot(jnp.tanh(pre_mu), wmufc_ref[...],
                      preferred_element_type=jnp.float32) + bmufc_ref[...]
    attr_logv = jnp.dot(jnp.tanh(pre_var), wvarfc_ref[...],
                        preferred_element_type=jnp.float32) + bvarfc_ref[...]
    attrz_ref[...] = attr_mu + eps_attr_ref[...] * jnp.exp(0.5 * attr_logv)


def _layer1_body(adj_ref, p_ref, w2_ref, q_ref):
    a = adj_ref[...].astype(p_ref.dtype)
    h = jnp.maximum(
        jnp.dot(a, p_ref[...], preferred_element_type=jnp.float32),
        0.0)
    q = jnp.dot(h, w2_ref[...], preferred_element_type=jnp.float32)
    q_ref[...] = q.astype(q_ref.dtype)


def _layer2_body(adj_ref, q_ref, eps_ref, attrz_ref, out_ref):
    out = q_ref.shape[1] // 2
    a = adj_ref[...].astype(q_ref.dtype)
    o = jnp.dot(a, q_ref[...], preferred_element_type=jnp.float32)
    z = o[:, :out] + eps_ref[...] * jnp.exp(0.5 * o[:, out:])
    out_ref[...] = jax.lax.dot_general(z, attrz_ref[...],
                                       (((1,), (1,)), ((), ())),
                                       preferred_element_type=jnp.float32)


def kernel(x, adj, W_node_mu1, W_node_mu2, W_node_var1, W_node_var2,
           W_attr_mu1, b_attr_mu1, W_attr_mu_fc, b_attr_mu_fc,
           W_attr_var1, b_attr_var1, W_attr_var_fc, b_attr_var_fc):
    n = adj.shape[0]
    f = x.shape[1]
    out = W_node_mu2.shape[0]
    f32 = jnp.float32

    nk = jax.random.key(123)
    nks = jax.random.split(nk, 4)
    node_noise = jax.random.bernoulli(
        nks[0], 0.5, (n, _S, _NOISE)).astype(f32)[:, 0, :]
    attr_noise = jax.random.bernoulli(
        nks[1], 0.5, (f, _S, _NOISE)).astype(f32)[:, 0, :]
    eps_node = jax.random.normal(nks[2], (n, 1, out), f32)[:, 0, :]
    eps_attr = jax.random.normal(nks[3], (f, 1, out), f32)[:, 0, :]

    wa = jnp.concatenate([W_node_mu1[_NOISE:], W_node_var1], axis=1)
    wnn = jnp.zeros((8, 2 * out), f32).at[:_NOISE, :out].set(W_node_mu1[:_NOISE])
    nn_pad = jnp.zeros((n, 8), f32).at[:, :_NOISE].set(node_noise)
    wb = jnp.concatenate([W_attr_mu1[_NOISE:], W_attr_var1], axis=1)
    wna = jnp.zeros((8, out), f32).at[:_NOISE].set(W_attr_mu1[:_NOISE])
    an_pad = jnp.zeros((f, 8), f32).at[:, :_NOISE].set(attr_noise)
    w2 = (jnp.zeros((2 * out, 2 * out), f32)
          .at[:out, :out].set(W_node_mu2)
          .at[out:, out:].set(W_node_var2))

    pcat, attr_z = pl.pallas_call(
        _prologue_body,
        out_shape=[jax.ShapeDtypeStruct((n, 2 * out), jnp.bfloat16),
                   jax.ShapeDtypeStruct((f, out), f32)],
    )(x, wa, nn_pad, wnn, wb, an_pad, wna,
      b_attr_mu1.reshape(1, -1), W_attr_mu_fc, b_attr_mu_fc.reshape(1, -1),
      b_attr_var1.reshape(1, -1), W_attr_var_fc, b_attr_var_fc.reshape(1, -1),
      eps_attr)

    bm = 400
    qcat = pl.pallas_call(
        _layer1_body,
        grid=(n // bm,),
        in_specs=[pl.BlockSpec((bm, n), lambda i: (i, 0)),
                  pl.BlockSpec((n, 2 * out), lambda i: (0, 0)),
                  pl.BlockSpec((2 * out, 2 * out), lambda i: (0, 0))],
        out_specs=pl.BlockSpec((bm, 2 * out), lambda i: (i, 0)),
        out_shape=jax.ShapeDtypeStruct((n, 2 * out), jnp.bfloat16),
        compiler_params=pltpu.CompilerParams(
            dimension_semantics=("parallel",)),
    )(adj, pcat, w2)

    recon = pl.pallas_call(
        _layer2_body,
        grid=(n // bm,),
        in_specs=[pl.BlockSpec((bm, n), lambda i: (i, 0)),
                  pl.BlockSpec((n, 2 * out), lambda i: (0, 0)),
                  pl.BlockSpec((bm, out), lambda i: (i, 0)),
                  pl.BlockSpec((f, out), lambda i: (0, 0))],
        out_specs=pl.BlockSpec((bm, f), lambda i: (i, 0)),
        out_shape=jax.ShapeDtypeStruct((n, f), f32),
        compiler_params=pltpu.CompilerParams(
            dimension_semantics=("parallel",)),
    )(adj, qcat, eps_node, attr_z)

    return recon
